# Initial kernel scaffold; baseline (speedup 1.0000x reference)
import jax
import jax.numpy as jnp
from jax.experimental import pallas as pl


def kernel(x, edge_index, W1, b1, W2, b2):
    raise NotImplementedError("write your pallas kernel here")


# speedup vs baseline: 2.4093x; 2.4093x over previous
import functools

import jax
import jax.numpy as jnp
from jax import lax
from jax.experimental import pallas as pl
from jax.experimental.pallas import tpu as pltpu
from jax.experimental.pallas import tpu_sc as plsc

N = 10000
HALF = N // 2
ACC_ROWS = 5120
TPT = ACC_ROWS // 16
CH = 128
NTILES = 16
OCH = (128, 128, 56)
OCH_LAST = (128, 128, 64)


def _copy_out(acc, out_hbm, c, s):
    lo = c * HALF + s * 312

    @pl.when(s < NTILES - 1)
    def _():
        off = 0
        for n in OCH:
            pltpu.sync_copy(acc.at[pl.ds(s * 312 + off, n)],
                            out_hbm.at[pl.ds(lo + off, n)])
            off += n

    @pl.when(s == NTILES - 1)
    def _():
        off = 0
        for n in OCH_LAST:
            pltpu.sync_copy(acc.at[pl.ds(s * 312 + off, n)],
                            out_hbm.at[pl.ds(lo + off, n)])
            off += n


def _local_idx(dstv, idxw, lo):
    for k in range(CH // 16):
        d = dstv[pl.ds(k * 16, 16)]
        loc = d - lo
        m = (loc >= 0) & (loc < HALF)
        idxw[pl.ds(k * 16, 16)] = jnp.where(m, loc, HALF + (d & 63))


def _make_deg(nch):
    mesh = plsc.VectorSubcoreMesh(core_axis_name="c", subcore_axis_name="s")

    @functools.partial(
        pl.kernel,
        mesh=mesh,
        out_type=jax.ShapeDtypeStruct((N, 128), jnp.float32),
        scratch_types=[
            pltpu.VMEM((CH,), jnp.int32),
            pltpu.VMEM((CH,), jnp.int32),
            pltpu.VMEM((CH, 128), jnp.float32),
            pltpu.VMEM_SHARED((ACC_ROWS, 128), jnp.float32),
            pltpu.SemaphoreType.DMA,
        ],
    )
    def deg(dst_hbm, ones_hbm, zero_hbm, out_hbm, dstv, idxw, ones_v, acc, sem):
        c = lax.axis_index("c")
        s = lax.axis_index("s")
        lo = c * HALF
        pltpu.sync_copy(ones_hbm, ones_v)
        pltpu.sync_copy(zero_hbm, acc.at[pl.ds(s * TPT, TPT)])
        plsc.subcore_barrier()

        def body(j, carry):
            base = s * (nch * CH) + j * CH
            pltpu.sync_copy(dst_hbm.at[pl.ds(base, CH)], dstv)
            _local_idx(dstv, idxw, lo)
            pltpu.async_copy(ones_v, acc.at[idxw], sem, add=True).wait()
            return carry

        lax.fori_loop(0, nch, body, 0)
        plsc.subcore_barrier()
        _copy_out(acc, out_hbm, c, s)

    return deg


def _make_agg(nch):
    mesh = plsc.VectorSubcoreMesh(core_axis_name="c", subcore_axis_name="s")

    @functools.partial(
        pl.kernel,
        mesh=mesh,
        out_type=[jax.ShapeDtypeStruct((N, 128), jnp.float32),
                  jax.ShapeDtypeStruct((N, 128), jnp.float32)],
        scratch_types=[
            pltpu.VMEM((CH,), jnp.int32),
            pltpu.VMEM((CH,), jnp.int32),
            pltpu.VMEM((CH,), jnp.int32),
            pltpu.VMEM((CH, 256), jnp.float32),
            pltpu.VMEM_SHARED((ACC_ROWS, 128), jnp.float32),
            pltpu.VMEM_SHARED((ACC_ROWS, 128), jnp.float32),
            pltpu.SemaphoreType.DMA,
            pltpu.SemaphoreType.DMA,
        ],
    )
    def agg(u_hbm, src_hbm, dst_hbm, zero_hbm, oa_hbm, ob_hbm,
            srcv, dstv, idxw, rows, acc0, acc1, gsem, ssem):
        c = lax.axis_index("c")
        s = lax.axis_index("s")
        lo = c * HALF
        pltpu.sync_copy(zero_hbm, acc0.at[pl.ds(s * TPT, TPT)])
        pltpu.sync_copy(zero_hbm, acc1.at[pl.ds(s * TPT, TPT)])
        plsc.subcore_barrier()

        def body(j, carry):
            base = s * (nch * CH) + j * CH
            pltpu.sync_copy(src_hbm.at[pl.ds(base, CH)], srcv)
            pltpu.sync_copy(dst_hbm.at[pl.ds(base, CH)], dstv)
            gcp = pltpu.async_copy(u_hbm.at[srcv], rows, gsem)
            _local_idx(dstv, idxw, lo)
            gcp.wait()
            pltpu.async_copy(rows.at[:, pl.ds(0, 128)], acc0.at[idxw],
                             ssem, add=True).wait()
            pltpu.async_copy(rows.at[:, pl.ds(128, 128)], acc1.at[idxw],
                             ssem, add=True).wait()
            return carry

        lax.fori_loop(0, nch, body, 0)
        plsc.subcore_barrier()
        _copy_out(acc0, oa_hbm, c, s)
        _copy_out(acc1, ob_hbm, c, s)

    return agg


BM = 1000


def _scale_body(deg_ref, x_ref, o_ref):
    dinv = lax.rsqrt(deg_ref[...][:, 0:1] + 1.0)
    o_ref[...] = x_ref[...] * dinv


def _l1_body(deg_ref, a0_ref, a1_ref, xp_ref, w_ref, b_ref, oa_ref, ob_ref):
    dinv = lax.rsqrt(deg_ref[...][:, 0:1] + 1.0)
    xp = xp_ref[...]
    z0 = (a0_ref[...] + xp[:, :128]) * dinv
    z1 = (a1_ref[...] + xp[:, 128:]) * dinv
    w = w_ref[...]
    h = (jnp.dot(z0, w[:128, :], preferred_element_type=jnp.float32)
         + jnp.dot(z1, w[128:, :], preferred_element_type=jnp.float32)
         + b_ref[...])
    h = jnp.maximum(h, 0.0) * dinv
    oa_ref[...] = h[:, :256]
    ob_ref[...] = h[:, 256:]


def _l2_body(deg_ref, a0_ref, a1_ref, b0_ref, b1_ref, ha_ref, hb_ref,
             w_ref, bias_ref, o_ref):
    dinv = lax.rsqrt(deg_ref[...][:, 0:1] + 1.0)
    ha = ha_ref[...]
    hb = hb_ref[...]
    z0 = (a0_ref[...] + ha[:, :128]) * dinv
    z1 = (a1_ref[...] + ha[:, 128:]) * dinv
    z2 = (b0_ref[...] + hb[:, :128]) * dinv
    z3 = (b1_ref[...] + hb[:, 128:]) * dinv
    w = w_ref[...]
    o_ref[...] = (jnp.dot(z0, w[:128, :], preferred_element_type=jnp.float32)
                  + jnp.dot(z1, w[128:256, :], preferred_element_type=jnp.float32)
                  + jnp.dot(z2, w[256:384, :], preferred_element_type=jnp.float32)
                  + jnp.dot(z3, w[384:, :], preferred_element_type=jnp.float32)
                  + bias_ref[...])


def _row_spec(cols):
    return pl.BlockSpec((BM, cols), lambda i: (i, 0))


def _full_spec(r, cols):
    return pl.BlockSpec((r, cols), lambda i: (0, 0))


def _seg(u, src, dst):
    return jax.ops.segment_sum(u[src], dst, num_segments=N)


def kernel(x, edge_index, W1, b1, W2, b2):
    src = edge_index[0].astype(jnp.int32)
    dst = edge_index[1].astype(jnp.int32)

    deg = _seg(jnp.ones((N, 128), jnp.float32), src, dst)

    grid = N // BM
    xp = pl.pallas_call(
        _scale_body,
        grid=(grid,),
        in_specs=[_row_spec(128), _row_spec(256)],
        out_specs=_row_spec(256),
        out_shape=jax.ShapeDtypeStruct((N, 256), jnp.float32),
    )(deg, x)

    agg1 = _seg(xp, src, dst)
    a0, a1 = agg1[:, :128], agg1[:, 128:]

    h1a, h1b = pl.pallas_call(
        _l1_body,
        grid=(grid,),
        in_specs=[_row_spec(128), _row_spec(128), _row_spec(128),
                  _row_spec(256), _full_spec(256, 512), _full_spec(1, 512)],
        out_specs=[_row_spec(256), _row_spec(256)],
        out_shape=[jax.ShapeDtypeStruct((N, 256), jnp.float32),
                   jax.ShapeDtypeStruct((N, 256), jnp.float32)],
    )(deg, a0, a1, xp, W1, b1.reshape(1, 512))

    agg2a = _seg(h1a, src, dst)
    agg2b = _seg(h1b, src, dst)

    out = pl.pallas_call(
        _l2_body,
        grid=(grid,),
        in_specs=[_row_spec(128), _row_spec(128), _row_spec(128),
                  _row_spec(128), _row_spec(128),
                  _row_spec(256), _row_spec(256),
                  _full_spec(512, 512), _full_spec(1, 512)],
        out_specs=_row_spec(512),
        out_shape=jax.ShapeDtypeStruct((N, 512), jnp.float32),
    )(deg, agg2a[:, :128], agg2a[:, 128:], agg2b[:, :128], agg2b[:, 128:],
      h1a, h1b, W2, b2.reshape(1, 512))
    return out
